# baseline (speedup 1.0000x reference)
import functools

import jax
import jax.numpy as jnp
from jax import lax
from jax.experimental import pallas as pl
from jax.experimental.pallas import tpu as pltpu
from jax.experimental.pallas import tpu_sc as plsc

N_TOKENS = 32 * 1024
K_CODES = 8192
D = 256

BN = 128
BKC = 2048
NB = N_TOKENS // BN


def _argmin_body(z_ref, emb_ref, idx_ref, mind_ref, loss_ref, enorm_s, embbf_s,
                 lsum_s):
    i = pl.program_id(0)
    flat = z_ref[0]

    @pl.when(i == 0)
    def _():
        emb = emb_ref[...]
        enorm_s[...] = jnp.sum(emb * emb, axis=1)[None, :]
        embbf_s[...] = emb.astype(jnp.bfloat16)

    scores2 = lax.dot_general(
        (flat * (-2.0)).astype(jnp.bfloat16), embbf_s[...],
        (((1,), (1,)), ((), ())),
        preferred_element_type=jnp.float32,
        precision=lax.Precision.DEFAULT)
    znorm = jnp.sum(flat * flat, axis=1)
    dist = (znorm[:, None] + scores2) + enorm_s[...]
    m = jnp.min(dist, axis=1)
    bidx = jnp.argmin(dist, axis=1).astype(jnp.int32)
    idx_ref[0, 0, :] = bidx
    mind_ref[0, 0, :] = m

    prev = jnp.where(i == 0, jnp.zeros((1, BN), jnp.float32), lsum_s[...])
    lsum_s[...] = prev + m[None, :]

    @pl.when(i == NB - 1)
    def _():
        loss_ref[0, 0] = jnp.sum(lsum_s[...])


_argmin_call = pl.pallas_call(
    _argmin_body,
    grid=(NB,),
    in_specs=[
        pl.BlockSpec((1, BN, D), lambda i: (i, 0, 0)),
        pl.BlockSpec((K_CODES, D), lambda i: (0, 0)),
    ],
    out_specs=[
        pl.BlockSpec((1, 1, BN), lambda i: (i, 0, 0)),
        pl.BlockSpec((1, 1, BN), lambda i: (i, 0, 0)),
        pl.BlockSpec(memory_space=pltpu.SMEM),
    ],
    out_shape=[
        jax.ShapeDtypeStruct((NB, 1, BN), jnp.int32),
        jax.ShapeDtypeStruct((NB, 1, BN), jnp.float32),
        jax.ShapeDtypeStruct((1, 1), jnp.float32),
    ],
    scratch_shapes=[
        pltpu.VMEM((1, K_CODES), jnp.float32),
        pltpu.VMEM((K_CODES, D), jnp.bfloat16),
        pltpu.VMEM((1, BN), jnp.float32),
    ],
)


_NC, _NS = 2, 16
_NW = _NC * _NS
_BPW = N_TOKENS // _NW
_CH = 128


def _gather_body(table_hbm, idx_hbm, out_hbm, idx_v, rows_a, rows_b,
                 gsem_a, gsem_b, wsem_a, wsem_b):
    wid = lax.axis_index("s") * _NC + lax.axis_index("c")
    base = wid * _BPW
    pltpu.sync_copy(idx_hbm.at[pl.ds(base, _BPW)], idx_v)
    bufs = (rows_a, rows_b)
    gsems = (gsem_a, gsem_b)
    wsems = (wsem_a, wsem_b)
    gcp = [None, None]
    wcp = [None, None]
    nch = _BPW // _CH
    for c in range(nch):
        b = c % 2
        if c >= 2:
            wcp[b].wait()
        gcp[b] = pltpu.async_copy(
            table_hbm.at[idx_v.at[pl.ds(c * _CH, _CH)]], bufs[b], gsems[b])
        if c >= 1:
            p = (c - 1) % 2
            gcp[p].wait()
            wcp[p] = pltpu.async_copy(
                bufs[p], out_hbm.at[pl.ds(base + (c - 1) * _CH, _CH)], wsems[p])
    last = (nch - 1) % 2
    gcp[last].wait()
    wcp[last] = pltpu.async_copy(
        bufs[last], out_hbm.at[pl.ds(base + (nch - 1) * _CH, _CH)], wsems[last])
    wcp[(nch - 2) % 2].wait()
    wcp[last].wait()


@functools.lru_cache(maxsize=1)
def _make_gather_call():
    return pl.kernel(
        _gather_body,
        out_type=jax.ShapeDtypeStruct((N_TOKENS, D), jnp.float32),
        scratch_types=[
            pltpu.VMEM((_BPW,), jnp.int32),
            pltpu.VMEM((_CH, D), jnp.float32),
            pltpu.VMEM((_CH, D), jnp.float32),
            pltpu.SemaphoreType.DMA,
            pltpu.SemaphoreType.DMA,
            pltpu.SemaphoreType.DMA,
            pltpu.SemaphoreType.DMA,
        ],
        mesh=plsc.VectorSubcoreMesh(
            core_axis_name="c", subcore_axis_name="s",
            num_cores=_NC, num_subcores=_NS),
    )


def kernel(z, embedding):
    zb = z.reshape(NB, BN, D)
    idx3, _mind, loss_acc = _argmin_call(zb, embedding)
    idx_flat = idx3.reshape(N_TOKENS)
    quant = _make_gather_call()(embedding, idx_flat).reshape(z.shape)
    loss = loss_acc[0, 0] * (1.0 + 0.25) / (N_TOKENS * D)
    return quant, loss, idx_flat.reshape(z.shape[:-1])


# speedup vs baseline: 1.2360x; 1.2360x over previous
import functools

import jax
import jax.numpy as jnp
from jax import lax
from jax.experimental import pallas as pl
from jax.experimental.pallas import tpu as pltpu
from jax.experimental.pallas import tpu_sc as plsc

N_TOKENS = 32 * 1024
K_CODES = 8192
D = 256

BN = 256
NB = N_TOKENS // BN


def _make_argmin_call(nb):
    def body(z_ref, emb_ref, idx_ref, mind_ref, loss_ref, enorm_s, embbf_s,
             lsum_s):
        i = pl.program_id(0)
        flat = z_ref[0]

        @pl.when(i == 0)
        def _():
            emb = emb_ref[...]
            enorm_s[...] = jnp.sum(emb * emb, axis=1)[None, :]
            embbf_s[...] = emb.astype(jnp.bfloat16)

        scores2 = lax.dot_general(
            (flat * (-2.0)).astype(jnp.bfloat16), embbf_s[...],
            (((1,), (1,)), ((), ())),
            preferred_element_type=jnp.float32,
            precision=lax.Precision.DEFAULT)
        znorm = jnp.sum(flat * flat, axis=1)
        dist = (znorm[:, None] + scores2) + enorm_s[...]
        m = jnp.min(dist, axis=1)
        bidx = jnp.argmin(dist, axis=1).astype(jnp.int32)
        idx_ref[0, 0, :] = bidx
        mind_ref[0, 0, :] = m

        prev = jnp.where(i == 0, jnp.zeros((1, BN), jnp.float32), lsum_s[...])
        lsum_s[...] = prev + m[None, :]

        @pl.when(i == nb - 1)
        def _():
            loss_ref[0, 0] = jnp.sum(lsum_s[...])

    return pl.pallas_call(
        body,
        grid=(nb,),
        in_specs=[
            pl.BlockSpec((1, BN, D), lambda i: (i, 0, 0)),
            pl.BlockSpec((K_CODES, D), lambda i: (0, 0)),
        ],
        out_specs=[
            pl.BlockSpec((1, 1, BN), lambda i: (i, 0, 0)),
            pl.BlockSpec((1, 1, BN), lambda i: (i, 0, 0)),
            pl.BlockSpec(memory_space=pltpu.SMEM),
        ],
        out_shape=[
            jax.ShapeDtypeStruct((nb, 1, BN), jnp.int32),
            jax.ShapeDtypeStruct((nb, 1, BN), jnp.float32),
            jax.ShapeDtypeStruct((1, 1), jnp.float32),
        ],
        scratch_shapes=[
            pltpu.VMEM((1, K_CODES), jnp.float32),
            pltpu.VMEM((K_CODES, D), jnp.bfloat16),
            pltpu.VMEM((1, BN), jnp.float32),
        ],
    )


_NC, _NS = 2, 16
_NW = _NC * _NS
_CH = 128


def _make_gather_body(bpw):
    def body(table_hbm, idx_hbm, out_hbm, idx_v, rows_a, rows_b,
             gsem_a, gsem_b, wsem_a, wsem_b):
        wid = lax.axis_index("s") * _NC + lax.axis_index("c")
        base = wid * bpw
        pltpu.sync_copy(idx_hbm.at[pl.ds(base, bpw)], idx_v)
        bufs = (rows_a, rows_b)
        gcp = [None, None]
        wcp = [None, None]
        nch = bpw // _CH
        for c in range(nch):
            b = c % 2
            if c >= 2:
                wcp[b].wait()
            gcp[b] = pltpu.async_copy(
                table_hbm.at[idx_v.at[pl.ds(c * _CH, _CH)]],
                bufs[b], (gsem_a, gsem_b)[b])
            if c >= 1:
                p = (c - 1) % 2
                gcp[p].wait()
                wcp[p] = pltpu.async_copy(
                    bufs[p], out_hbm.at[pl.ds(base + (c - 1) * _CH, _CH)],
                    (wsem_a, wsem_b)[p])
        last = (nch - 1) % 2
        gcp[last].wait()
        wcp[last] = pltpu.async_copy(
            bufs[last], out_hbm.at[pl.ds(base + (nch - 1) * _CH, _CH)],
            (wsem_a, wsem_b)[last])
        wcp[(nch - 2) % 2].wait()
        wcp[last].wait()

    return body


@functools.lru_cache(maxsize=2)
def _make_gather_call(ntok):
    bpw = ntok // _NW
    return pl.kernel(
        _make_gather_body(bpw),
        out_type=jax.ShapeDtypeStruct((ntok, D), jnp.float32),
        scratch_types=[
            pltpu.VMEM((bpw,), jnp.int32),
            pltpu.VMEM((_CH, D), jnp.float32),
            pltpu.VMEM((_CH, D), jnp.float32),
            pltpu.SemaphoreType.DMA,
            pltpu.SemaphoreType.DMA,
            pltpu.SemaphoreType.DMA,
            pltpu.SemaphoreType.DMA,
        ],
        mesh=plsc.VectorSubcoreMesh(
            core_axis_name="c", subcore_axis_name="s",
            num_cores=_NC, num_subcores=_NS),
    )


def kernel(z, embedding):
    zb = z.reshape(NB, BN, D)
    half_nb = NB // 2
    half_tok = N_TOKENS // 2
    argmin_half = _make_argmin_call(half_nb)
    gather_half = _make_gather_call(half_tok)

    idx_a, _mind_a, loss_a = argmin_half(zb[:half_nb], embedding)
    quant_a = gather_half(embedding, idx_a.reshape(half_tok))
    idx_b, _mind_b, loss_b = argmin_half(zb[half_nb:], embedding)
    quant_b = gather_half(embedding, idx_b.reshape(half_tok))

    quant = jnp.concatenate([quant_a, quant_b], axis=0).reshape(z.shape)
    idx_flat = jnp.concatenate(
        [idx_a.reshape(half_tok), idx_b.reshape(half_tok)])
    loss = (loss_a[0, 0] + loss_b[0, 0]) * (1.0 + 0.25) / (N_TOKENS * D)
    return quant, loss, idx_flat.reshape(z.shape[:-1])


# speedup vs baseline: 1.3950x; 1.1287x over previous
import functools

import jax
import jax.numpy as jnp
from jax import lax
from jax.experimental import pallas as pl
from jax.experimental.pallas import tpu as pltpu
from jax.experimental.pallas import tpu_sc as plsc

N_TOKENS = 32 * 1024
K_CODES = 8192
D = 256

BN = 256
NB = N_TOKENS // BN


def _argmin_body(z_ref, emb_ref, idx_ref, loss_ref, enorm_s, embbf_s):
    i = pl.program_id(0)
    flat = z_ref[0]

    @pl.when(i == 0)
    def _():
        emb = emb_ref[...]
        enorm_s[...] = jnp.sum(emb * emb, axis=1)[None, :]
        embbf_s[...] = emb.astype(jnp.bfloat16)

    scores2 = lax.dot_general(
        (flat * (-2.0)).astype(jnp.bfloat16), embbf_s[...],
        (((1,), (1,)), ((), ())),
        preferred_element_type=jnp.float32,
        precision=lax.Precision.DEFAULT)
    znorm = jnp.sum(flat * flat, axis=1)
    dist = (znorm[:, None] + scores2) + enorm_s[...]
    m = jnp.min(dist, axis=1)
    bidx = jnp.argmin(dist, axis=1).astype(jnp.int32)
    idx_ref[0, 0, :] = bidx

    prev = jnp.where(i == 0, 0.0, loss_ref[0, 0])
    loss_ref[0, 0] = prev + jnp.sum(m)


_argmin_call = pl.pallas_call(
    _argmin_body,
    grid=(NB,),
    in_specs=[
        pl.BlockSpec((1, BN, D), lambda i: (i, 0, 0)),
        pl.BlockSpec((K_CODES, D), lambda i: (0, 0)),
    ],
    out_specs=[
        pl.BlockSpec((1, 1, BN), lambda i: (i, 0, 0)),
        pl.BlockSpec(memory_space=pltpu.SMEM),
    ],
    out_shape=[
        jax.ShapeDtypeStruct((NB, 1, BN), jnp.int32),
        jax.ShapeDtypeStruct((1, 1), jnp.float32),
    ],
    scratch_shapes=[
        pltpu.VMEM((1, K_CODES), jnp.float32),
        pltpu.VMEM((K_CODES, D), jnp.bfloat16),
    ],
)


_NC, _NS = 2, 16
_NW = _NC * _NS
_BPW = N_TOKENS // _NW
_CH = 128


def _gather_body(table_hbm, idx_hbm, out_hbm, idx_v, rows_a, rows_b,
                 sem_a, sem_b):
    wid = lax.axis_index("s") * _NC + lax.axis_index("c")
    base = wid * _BPW
    pltpu.sync_copy(idx_hbm.at[pl.ds(base, _BPW)], idx_v)
    bufs = (rows_a, rows_b)
    sems = (sem_a, sem_b)
    copies = [None, None]
    nch = _BPW // _CH
    for c in range(nch):
        copies[c % 2] = pltpu.async_copy(
            table_hbm.at[idx_v.at[pl.ds(c * _CH, _CH)]],
            bufs[c % 2], sems[c % 2])
        if c > 0:
            copies[(c - 1) % 2].wait()
            pltpu.sync_copy(bufs[(c - 1) % 2],
                            out_hbm.at[pl.ds(base + (c - 1) * _CH, _CH)])
    copies[(nch - 1) % 2].wait()
    pltpu.sync_copy(bufs[(nch - 1) % 2],
                    out_hbm.at[pl.ds(base + (nch - 1) * _CH, _CH)])


@functools.lru_cache(maxsize=1)
def _make_gather_call():
    return pl.kernel(
        _gather_body,
        out_type=jax.ShapeDtypeStruct((N_TOKENS, D), jnp.float32),
        scratch_types=[
            pltpu.VMEM((_BPW,), jnp.int32),
            pltpu.VMEM((_CH, D), jnp.float32),
            pltpu.VMEM((_CH, D), jnp.float32),
            pltpu.SemaphoreType.DMA,
            pltpu.SemaphoreType.DMA,
        ],
        mesh=plsc.VectorSubcoreMesh(
            core_axis_name="c", subcore_axis_name="s",
            num_cores=_NC, num_subcores=_NS),
    )


def kernel(z, embedding):
    zb = z.reshape(NB, BN, D)
    idx3, loss_acc = _argmin_call(zb, embedding)
    idx_flat = idx3.reshape(N_TOKENS)
    quant = _make_gather_call()(embedding, idx_flat).reshape(z.shape)
    loss = loss_acc[0, 0] * (1.0 + 0.25) / (N_TOKENS * D)
    return quant, loss, idx_flat.reshape(z.shape[:-1])


# speedup vs baseline: 1.3987x; 1.0027x over previous
import functools

import jax
import jax.numpy as jnp
from jax import lax
from jax.experimental import pallas as pl
from jax.experimental.pallas import tpu as pltpu
from jax.experimental.pallas import tpu_sc as plsc

N_TOKENS = 32 * 1024
K_CODES = 8192
D = 256

BN = 256
NB = N_TOKENS // BN


def _argmin_body(z_ref, emb_ref, idx_ref, loss_ref, enorm_s, embbf_s):
    i = pl.program_id(0)
    flat = z_ref[0]

    @pl.when(i == 0)
    def _():
        emb = emb_ref[...]
        enorm_s[...] = jnp.sum(emb * emb, axis=1)[None, :]
        embbf_s[...] = emb.astype(jnp.bfloat16)

    scores2 = lax.dot_general(
        (flat * (-2.0)).astype(jnp.bfloat16), embbf_s[...],
        (((1,), (1,)), ((), ())),
        preferred_element_type=jnp.float32,
        precision=lax.Precision.DEFAULT)
    znorm = jnp.sum(flat * flat, axis=1)
    dist = (znorm[:, None] + scores2) + enorm_s[...]
    m = jnp.min(dist, axis=1)
    bidx = jnp.argmin(dist, axis=1).astype(jnp.int32)
    idx_ref[0, 0, :] = bidx

    prev = jnp.where(i == 0, 0.0, loss_ref[0, 0])
    loss_ref[0, 0] = prev + jnp.sum(m)


_argmin_call = pl.pallas_call(
    _argmin_body,
    grid=(NB,),
    in_specs=[
        pl.BlockSpec((1, BN, D), lambda i: (i, 0, 0)),
        pl.BlockSpec((K_CODES, D), lambda i: (0, 0)),
    ],
    out_specs=[
        pl.BlockSpec((1, 1, BN), lambda i: (i, 0, 0)),
        pl.BlockSpec(memory_space=pltpu.SMEM),
    ],
    out_shape=[
        jax.ShapeDtypeStruct((NB, 1, BN), jnp.int32),
        jax.ShapeDtypeStruct((1, 1), jnp.float32),
    ],
    scratch_shapes=[
        pltpu.VMEM((1, K_CODES), jnp.float32),
        pltpu.VMEM((K_CODES, D), jnp.bfloat16),
    ],
)


_NC, _NS = 2, 16
_NW = _NC * _NS
_BPW = N_TOKENS // _NW
_CH = 128


def _gather_body(table_hbm, idx_hbm, out_hbm, idx_v, rows_a, rows_b, rows_c,
                 sem_a, sem_b, sem_c):
    wid = lax.axis_index("s") * _NC + lax.axis_index("c")
    base = wid * _BPW
    pltpu.sync_copy(idx_hbm.at[pl.ds(base, _BPW)], idx_v)
    bufs = (rows_a, rows_b, rows_c)
    sems = (sem_a, sem_b, sem_c)
    copies = [None, None, None]
    nch = _BPW // _CH
    for c in range(nch):
        copies[c % 3] = pltpu.async_copy(
            table_hbm.at[idx_v.at[pl.ds(c * _CH, _CH)]],
            bufs[c % 3], sems[c % 3])
        if c > 0:
            copies[(c - 1) % 3].wait()
            pltpu.sync_copy(bufs[(c - 1) % 3],
                            out_hbm.at[pl.ds(base + (c - 1) * _CH, _CH)])
    copies[(nch - 1) % 3].wait()
    pltpu.sync_copy(bufs[(nch - 1) % 3],
                    out_hbm.at[pl.ds(base + (nch - 1) * _CH, _CH)])


@functools.lru_cache(maxsize=1)
def _make_gather_call():
    return pl.kernel(
        _gather_body,
        out_type=jax.ShapeDtypeStruct((N_TOKENS, D), jnp.float32),
        scratch_types=[
            pltpu.VMEM((_BPW,), jnp.int32),
            pltpu.VMEM((_CH, D), jnp.float32),
            pltpu.VMEM((_CH, D), jnp.float32),
            pltpu.VMEM((_CH, D), jnp.float32),
            pltpu.SemaphoreType.DMA,
            pltpu.SemaphoreType.DMA,
            pltpu.SemaphoreType.DMA,
        ],
        mesh=plsc.VectorSubcoreMesh(
            core_axis_name="c", subcore_axis_name="s",
            num_cores=_NC, num_subcores=_NS),
    )


def kernel(z, embedding):
    zb = z.reshape(NB, BN, D)
    idx3, loss_acc = _argmin_call(zb, embedding)
    idx_flat = idx3.reshape(N_TOKENS)
    quant = _make_gather_call()(embedding, idx_flat).reshape(z.shape)
    loss = loss_acc[0, 0] * (1.0 + 0.25) / (N_TOKENS * D)
    return quant, loss, idx_flat.reshape(z.shape[:-1])
